# baseline (speedup 1.0000x reference)
import functools

import jax
import jax.numpy as jnp
from jax import lax
from jax.experimental import pallas as pl
from jax.experimental.pallas import tpu as pltpu
from jax.experimental.pallas import tpu_sc as plsc

K_NEG = 75
GAMMA = 1.0
N_NODES = 30000
D_FEAT = 64
T_LINKS = 1024
N_ANCH = 2 * T_LINKS
ROW_TILE = 256
N_TILES = N_ANCH // ROW_TILE
BS_ITERS = 7

_NC = 2
_NS = 16
_NW = _NC * _NS
_B_PER_W = N_ANCH // _NW
_P_PER_W = _B_PER_W // 2

@functools.lru_cache(maxsize=1)
def _sc_gather_pos_fn():
    mesh = plsc.VectorSubcoreMesh(
        core_axis_name="c", subcore_axis_name="s",
        num_cores=_NC, num_subcores=_NS)

    @functools.partial(
        pl.kernel,
        mesh=mesh,
        out_type=[
            jax.ShapeDtypeStruct((N_ANCH, D_FEAT), jnp.float32),
            jax.ShapeDtypeStruct((N_ANCH,), jnp.float32),
        ],
        scratch_types=[
            pltpu.VMEM((_B_PER_W,), jnp.int32),
            pltpu.VMEM((_B_PER_W, D_FEAT), jnp.float32),
            pltpu.VMEM((_B_PER_W,), jnp.float32),
            pltpu.SemaphoreType.DMA,
        ],
        compiler_params=pltpu.CompilerParams(
            needs_layout_passes=False, use_tc_tiling_on_sc=False),
    )
    def _sc_gather_pos(table_hbm, idx_hbm, a_hbm, dv_hbm,
                       idx_v, rows_v, dv_v, sem):
        wid = lax.axis_index("s") * _NC + lax.axis_index("c")
        base = wid * _B_PER_W
        pltpu.sync_copy(idx_hbm.at[pl.ds(base, _B_PER_W)], idx_v)
        pltpu.async_copy(table_hbm.at[idx_v], rows_v, sem).wait()
        for g in range(_P_PER_W // 16):
            pids = lax.iota(jnp.int32, 16) + g * 16
            li = 2 * pids
            ri = li + 1
            acc = jnp.zeros((16,), jnp.float32)
            for f in range(D_FEAT):
                fv = jnp.full((16,), f, jnp.int32)
                lv = plsc.load_gather(rows_v, [li, fv])
                rv = plsc.load_gather(rows_v, [ri, fv])
                dd = lv - rv
                acc = acc + dd * dd
            dvg = acc + GAMMA
            plsc.store_scatter(dv_v, [li], dvg)
            plsc.store_scatter(dv_v, [ri], dvg)
        pltpu.sync_copy(rows_v, a_hbm.at[pl.ds(base, _B_PER_W)])
        pltpu.sync_copy(dv_v, dv_hbm.at[pl.ds(base, _B_PER_W)])

    return _sc_gather_pos


def _tc_body(a_ref, dv_ref, e_ref, out_ref, e2_ref):
    step = pl.program_id(0)
    kf = jnp.float32(K_NEG)

    @pl.when(step == 0)
    def _init():
        e = e_ref[...]
        e2_ref[...] = lax.dot_general(
            jnp.ones((8, D_FEAT), jnp.float32), e * e,
            (((1,), (1,)), ((), ())))
        out_ref[...] = jnp.zeros((1, 1), jnp.float32)

    a = a_ref[...]
    a2 = jnp.sum(a * a, axis=1, keepdims=True)
    cross = lax.dot_general(a * -2.0, e_ref[...], (((1,), (1,)), ((), ())),
                            precision=lax.Precision.DEFAULT)
    e2row = e2_ref[0:1, :]
    dist = (a2 + e2row) + cross

    e2max = jnp.max(e2row, axis=1, keepdims=True)
    lo0 = jnp.full((ROW_TILE, 1), -1.0, jnp.float32)
    hi0 = a2 + e2max + 2.0 * jnp.sqrt(a2 * e2max) + 1.0
    c0 = jnp.zeros((ROW_TILE, 1), jnp.float32)
    ch0 = jnp.full((ROW_TILE, 1), float(N_NODES), jnp.float32)

    def body(_, carry):
        lo, hi, clo, chi = carry
        mid = 0.5 * (lo + hi)
        parts = [
            jnp.sum((dist[:, 0:7680] < mid).astype(jnp.float32),
                    axis=1, keepdims=True),
            jnp.sum((dist[:, 7680:15360] < mid).astype(jnp.float32),
                    axis=1, keepdims=True),
            jnp.sum((dist[:, 15360:23040] < mid).astype(jnp.float32),
                    axis=1, keepdims=True),
            jnp.sum((dist[:, 23040:] < mid).astype(jnp.float32),
                    axis=1, keepdims=True),
        ]
        cnt = (parts[0] + parts[1]) + (parts[2] + parts[3])
        ge = cnt >= kf
        return (jnp.where(ge, lo, mid), jnp.where(ge, mid, hi),
                jnp.where(ge, clo, cnt), jnp.where(ge, cnt, chi))

    lo, hi, cnt_lo, cnt_hi = lax.fori_loop(
        0, BS_ITERS, body, (lo0, hi0, c0, ch0))

    d = dv_ref[...]
    mask = dist < lo
    s_sel = jnp.sum(
        jnp.where(mask, jnp.maximum(d - dist, 0.0), 0.0),
        axis=1, keepdims=True)
    frac = (kf - cnt_lo) / jnp.maximum(cnt_hi - cnt_lo, 1.0)
    tau_est = lo + frac * (hi - lo)
    vbar = 0.5 * (lo + tau_est)
    row_s = s_sel + (kf - cnt_lo) * jnp.maximum(d - vbar, 0.0)
    tile_sum = jnp.sum(row_s, keepdims=True) * (1.0 / (2.0 * K_NEG * T_LINKS))
    out_ref[...] += tile_sum


_tc_call = pl.pallas_call(
    _tc_body,
    grid=(N_TILES,),
    in_specs=[
        pl.BlockSpec((ROW_TILE, D_FEAT), lambda i: (i, 0)),
        pl.BlockSpec((ROW_TILE, 1), lambda i: (i, 0)),
        pl.BlockSpec((N_NODES, D_FEAT), lambda i: (0, 0)),
    ],
    out_specs=pl.BlockSpec((1, 1), lambda i: (0, 0)),
    out_shape=jax.ShapeDtypeStruct((1, 1), jnp.float32),
    scratch_shapes=[pltpu.VMEM((8, N_NODES), jnp.float32)],
)


def kernel(embeddings, train_links):
    idx = jnp.reshape(train_links, (N_ANCH,))
    a, dv = _sc_gather_pos_fn()(embeddings, idx)
    out = _tc_call(a, jnp.reshape(dv, (N_ANCH, 1)), embeddings)
    return out[0, 0]


# speedup vs baseline: 1.0130x; 1.0130x over previous
import functools

import jax
import jax.numpy as jnp
from jax import lax
from jax.experimental import pallas as pl
from jax.experimental.pallas import tpu as pltpu
from jax.experimental.pallas import tpu_sc as plsc

K_NEG = 75
GAMMA = 1.0
N_NODES = 30000
D_FEAT = 64
T_LINKS = 1024
N_ANCH = 2 * T_LINKS
ROW_TILE = 256
N_TILES = N_ANCH // ROW_TILE
BS_ITERS = 7

_NC = 2
_NS = 16
_NW = _NC * _NS
_B_PER_W = N_ANCH // _NW
_P_PER_W = _B_PER_W // 2

@functools.lru_cache(maxsize=1)
def _sc_gather_pos_fn():
    mesh = plsc.VectorSubcoreMesh(
        core_axis_name="c", subcore_axis_name="s",
        num_cores=_NC, num_subcores=_NS)

    @functools.partial(
        pl.kernel,
        mesh=mesh,
        out_type=[
            jax.ShapeDtypeStruct((N_ANCH, D_FEAT), jnp.float32),
            jax.ShapeDtypeStruct((N_ANCH,), jnp.float32),
        ],
        scratch_types=[
            pltpu.VMEM((_B_PER_W,), jnp.int32),
            pltpu.VMEM((_B_PER_W, D_FEAT), jnp.float32),
            pltpu.VMEM((_B_PER_W,), jnp.float32),
            pltpu.SemaphoreType.DMA,
        ],
        compiler_params=pltpu.CompilerParams(
            needs_layout_passes=False, use_tc_tiling_on_sc=False),
    )
    def _sc_gather_pos(table_hbm, idx_hbm, a_hbm, dv_hbm,
                       idx_v, rows_v, dv_v, sem):
        wid = lax.axis_index("s") * _NC + lax.axis_index("c")
        base = wid * _B_PER_W
        pltpu.sync_copy(idx_hbm.at[pl.ds(base, _B_PER_W)], idx_v)
        pltpu.async_copy(table_hbm.at[idx_v], rows_v, sem).wait()
        for g in range(_P_PER_W // 16):
            pids = lax.iota(jnp.int32, 16) + g * 16
            li = 2 * pids
            ri = li + 1
            acc = jnp.zeros((16,), jnp.float32)
            for f in range(D_FEAT):
                fv = jnp.full((16,), f, jnp.int32)
                lv = plsc.load_gather(rows_v, [li, fv])
                rv = plsc.load_gather(rows_v, [ri, fv])
                dd = lv - rv
                acc = acc + dd * dd
            dvg = acc + GAMMA
            plsc.store_scatter(dv_v, [li], dvg)
            plsc.store_scatter(dv_v, [ri], dvg)
        pltpu.sync_copy(rows_v, a_hbm.at[pl.ds(base, _B_PER_W)])
        pltpu.sync_copy(dv_v, dv_hbm.at[pl.ds(base, _B_PER_W)])

    return _sc_gather_pos


def _tc_body(a_ref, dv_ref, e_ref, out_ref, e2_ref):
    step = pl.program_id(0)
    kf = jnp.float32(K_NEG)

    @pl.when(step == 0)
    def _init():
        e = e_ref[...]
        e2_ref[...] = lax.dot_general(
            jnp.ones((8, D_FEAT), jnp.float32), e * e,
            (((1,), (1,)), ((), ())))
        out_ref[...] = jnp.zeros((1, 1), jnp.float32)

    a = a_ref[...]
    a2 = jnp.sum(a * a, axis=1, keepdims=True)
    cross = lax.dot_general(a * -2.0, e_ref[...], (((1,), (1,)), ((), ())),
                            precision=lax.Precision.DEFAULT)
    e2row = e2_ref[0:1, :]
    dist = (a2 + e2row) + cross

    e2max = jnp.max(e2row, axis=1, keepdims=True)
    lo0 = jnp.full((ROW_TILE, 1), -1.0, jnp.float32)
    hi0 = a2 + e2max + 2.0 * jnp.sqrt(a2 * e2max) + 1.0
    c0 = jnp.zeros((ROW_TILE, 1), jnp.float32)
    ch0 = jnp.full((ROW_TILE, 1), float(N_NODES), jnp.float32)

    def body(_, carry):
        lo, hi, clo, chi = carry
        mid = 0.5 * (lo + hi)
        parts = [
            jnp.sum((dist[:, 0:7680] < mid).astype(jnp.float32),
                    axis=1, keepdims=True),
            jnp.sum((dist[:, 7680:15360] < mid).astype(jnp.float32),
                    axis=1, keepdims=True),
            jnp.sum((dist[:, 15360:23040] < mid).astype(jnp.float32),
                    axis=1, keepdims=True),
            jnp.sum((dist[:, 23040:] < mid).astype(jnp.float32),
                    axis=1, keepdims=True),
        ]
        cnt = (parts[0] + parts[1]) + (parts[2] + parts[3])
        ge = cnt >= kf
        return (jnp.where(ge, lo, mid), jnp.where(ge, mid, hi),
                jnp.where(ge, clo, cnt), jnp.where(ge, cnt, chi))

    lo, hi, cnt_lo, cnt_hi = lax.fori_loop(
        0, BS_ITERS, body, (lo0, hi0, c0, ch0))

    d = dv_ref[...]

    def _part(sl):
        dd = dist[:, sl]
        return jnp.sum(
            jnp.where(dd < lo, jnp.maximum(d - dd, 0.0), 0.0),
            axis=1, keepdims=True)

    s_sel = ((_part(slice(0, 7680)) + _part(slice(7680, 15360)))
             + (_part(slice(15360, 23040)) + _part(slice(23040, None))))
    frac = (kf - cnt_lo) / jnp.maximum(cnt_hi - cnt_lo, 1.0)
    tau_est = lo + frac * (hi - lo)
    vbar = 0.5 * (lo + tau_est)
    row_s = s_sel + (kf - cnt_lo) * jnp.maximum(d - vbar, 0.0)
    tile_sum = jnp.sum(row_s, keepdims=True) * (1.0 / (2.0 * K_NEG * T_LINKS))
    out_ref[...] += tile_sum


_tc_call = pl.pallas_call(
    _tc_body,
    grid=(N_TILES,),
    in_specs=[
        pl.BlockSpec((ROW_TILE, D_FEAT), lambda i: (i, 0)),
        pl.BlockSpec((ROW_TILE, 1), lambda i: (i, 0)),
        pl.BlockSpec((N_NODES, D_FEAT), lambda i: (0, 0)),
    ],
    out_specs=pl.BlockSpec((1, 1), lambda i: (0, 0)),
    out_shape=jax.ShapeDtypeStruct((1, 1), jnp.float32),
    scratch_shapes=[pltpu.VMEM((8, N_NODES), jnp.float32)],
)


def kernel(embeddings, train_links):
    idx = jnp.reshape(train_links, (N_ANCH,))
    a, dv = _sc_gather_pos_fn()(embeddings, idx)
    out = _tc_call(a, jnp.reshape(dv, (N_ANCH, 1)), embeddings)
    return out[0, 0]


# speedup vs baseline: 1.0157x; 1.0026x over previous
import functools

import jax
import jax.numpy as jnp
from jax import lax
from jax.experimental import pallas as pl
from jax.experimental.pallas import tpu as pltpu
from jax.experimental.pallas import tpu_sc as plsc

K_NEG = 75
GAMMA = 1.0
N_NODES = 30000
D_FEAT = 64
T_LINKS = 1024
N_ANCH = 2 * T_LINKS
ROW_TILE = 256
N_TILES = N_ANCH // ROW_TILE
BS_ITERS = 7

_NC = 2
_NS = 16
_NW = _NC * _NS
_B_PER_W = N_ANCH // _NW
_P_PER_W = _B_PER_W // 2

@functools.lru_cache(maxsize=1)
def _sc_gather_pos_fn():
    mesh = plsc.VectorSubcoreMesh(
        core_axis_name="c", subcore_axis_name="s",
        num_cores=_NC, num_subcores=_NS)

    @functools.partial(
        pl.kernel,
        mesh=mesh,
        out_type=[
            jax.ShapeDtypeStruct((N_ANCH, D_FEAT), jnp.float32),
            jax.ShapeDtypeStruct((N_ANCH,), jnp.float32),
        ],
        scratch_types=[
            pltpu.VMEM((_B_PER_W,), jnp.int32),
            pltpu.VMEM((_B_PER_W, D_FEAT), jnp.float32),
            pltpu.VMEM((_B_PER_W,), jnp.float32),
            pltpu.SemaphoreType.DMA,
        ],
        compiler_params=pltpu.CompilerParams(
            needs_layout_passes=False, use_tc_tiling_on_sc=False),
    )
    def _sc_gather_pos(table_hbm, idx_hbm, a_hbm, dv_hbm,
                       idx_v, rows_v, dv_v, sem):
        wid = lax.axis_index("s") * _NC + lax.axis_index("c")
        base = wid * _B_PER_W
        pltpu.sync_copy(idx_hbm.at[pl.ds(base, _B_PER_W)], idx_v)
        pltpu.async_copy(table_hbm.at[idx_v], rows_v, sem).wait()
        for g in range(_P_PER_W // 16):
            pids = lax.iota(jnp.int32, 16) + g * 16
            li = 2 * pids
            ri = li + 1
            acc = jnp.zeros((16,), jnp.float32)
            for f in range(D_FEAT):
                fv = jnp.full((16,), f, jnp.int32)
                lv = plsc.load_gather(rows_v, [li, fv])
                rv = plsc.load_gather(rows_v, [ri, fv])
                dd = lv - rv
                acc = acc + dd * dd
            dvg = acc + GAMMA
            plsc.store_scatter(dv_v, [li], dvg)
            plsc.store_scatter(dv_v, [ri], dvg)
        pltpu.sync_copy(rows_v, a_hbm.at[pl.ds(base, _B_PER_W)])
        pltpu.sync_copy(dv_v, dv_hbm.at[pl.ds(base, _B_PER_W)])

    return _sc_gather_pos


def _tc_body(a_ref, dv_ref, e_ref, out_ref, e2_ref):
    step = pl.program_id(0)
    kf = jnp.float32(K_NEG)

    @pl.when(step == 0)
    def _init():
        e = e_ref[...]
        e2_ref[...] = lax.dot_general(
            jnp.ones((8, D_FEAT), jnp.float32), e * e,
            (((1,), (1,)), ((), ())))
        out_ref[...] = jnp.zeros((1, 1), jnp.float32)

    a = a_ref[...]
    a2 = jnp.sum(a * a, axis=1, keepdims=True)
    cross = lax.dot_general(a * -2.0, e_ref[...], (((1,), (1,)), ((), ())),
                            precision=lax.Precision.DEFAULT)
    e2row = e2_ref[0:1, :]
    dist = (a2 + e2row) + cross

    e2max = jnp.max(e2row, axis=1, keepdims=True)
    lo0 = jnp.full((ROW_TILE, 1), -1.0, jnp.float32)
    hi0 = a2 + e2max + 2.0 * jnp.sqrt(a2 * e2max) + 1.0
    c0 = jnp.zeros((ROW_TILE, 1), jnp.float32)
    ch0 = jnp.full((ROW_TILE, 1), float(N_NODES), jnp.float32)

    def body(_, carry):
        lo, hi, clo, chi = carry
        mid = 0.5 * (lo + hi)
        parts = [
            jnp.sum((dist[:, s * 3840:(s + 1) * 3840 if s < 7 else None]
                     < mid).astype(jnp.float32), axis=1, keepdims=True)
            for s in range(8)
        ]
        cnt = (((parts[0] + parts[1]) + (parts[2] + parts[3]))
               + ((parts[4] + parts[5]) + (parts[6] + parts[7])))
        ge = cnt >= kf
        return (jnp.where(ge, lo, mid), jnp.where(ge, mid, hi),
                jnp.where(ge, clo, cnt), jnp.where(ge, cnt, chi))

    lo, hi, cnt_lo, cnt_hi = lax.fori_loop(
        0, BS_ITERS, body, (lo0, hi0, c0, ch0))

    d = dv_ref[...]

    def _part(sl):
        dd = dist[:, sl]
        return jnp.sum(
            jnp.where(dd < lo, jnp.maximum(d - dd, 0.0), 0.0),
            axis=1, keepdims=True)

    s_sel = ((_part(slice(0, 7680)) + _part(slice(7680, 15360)))
             + (_part(slice(15360, 23040)) + _part(slice(23040, None))))
    frac = (kf - cnt_lo) / jnp.maximum(cnt_hi - cnt_lo, 1.0)
    tau_est = lo + frac * (hi - lo)
    vbar = 0.5 * (lo + tau_est)
    row_s = s_sel + (kf - cnt_lo) * jnp.maximum(d - vbar, 0.0)
    tile_sum = jnp.sum(row_s, keepdims=True) * (1.0 / (2.0 * K_NEG * T_LINKS))
    out_ref[...] += tile_sum


_tc_call = pl.pallas_call(
    _tc_body,
    grid=(N_TILES,),
    in_specs=[
        pl.BlockSpec((ROW_TILE, D_FEAT), lambda i: (i, 0)),
        pl.BlockSpec((ROW_TILE, 1), lambda i: (i, 0)),
        pl.BlockSpec((N_NODES, D_FEAT), lambda i: (0, 0)),
    ],
    out_specs=pl.BlockSpec((1, 1), lambda i: (0, 0)),
    out_shape=jax.ShapeDtypeStruct((1, 1), jnp.float32),
    scratch_shapes=[pltpu.VMEM((8, N_NODES), jnp.float32)],
)


def kernel(embeddings, train_links):
    idx = jnp.reshape(train_links, (N_ANCH,))
    a, dv = _sc_gather_pos_fn()(embeddings, idx)
    out = _tc_call(a, jnp.reshape(dv, (N_ANCH, 1)), embeddings)
    return out[0, 0]
